# Initial kernel scaffold; baseline (speedup 1.0000x reference)
import jax
import jax.numpy as jnp
from jax.experimental import pallas as pl


def kernel(x, edge_index, edge_weight, rnn_w_ih, rnn_w_hh, rnn_b_ih, rnn_b_hh, h, alpha0, w, dvec):
    raise NotImplementedError("write your pallas kernel here")


# speedup vs baseline: 2.4486x; 2.4486x over previous
import functools

import jax
import jax.numpy as jnp
from jax import lax
from jax.experimental import pallas as pl
from jax.experimental.pallas import tpu as pltpu
from jax.experimental.pallas import tpu_sc as plsc

N = 10000
DH = 128
E = 320000
NSTEPS = 20
DT = 0.9 / NSTEPS

NT = 16
K = 128
NCH = 160
EPT = NCH * K
EPAD = NT * EPT
NPAD = 10240
RPT = NPAD // NT
SCH = 32


def _spmm_body(sa, sb, srcg, dstg, wg, axa, axb,
               acc, sidx, didx, wall, rows, gsem):
    c = lax.axis_index("c")
    s = lax.axis_index("s")

    def zrow(e, carry):
        for l in range(DH // 16):
            rows[e, pl.ds(l * 16, 16)] = jnp.zeros((16,), jnp.float32)
        return carry
    lax.fori_loop(0, K, zrow, 0)

    base = s * RPT
    for j in range(RPT // K):
        pltpu.sync_copy(rows, acc.at[pl.ds(base + j * K, K)])
    rem = RPT % K
    if rem:
        pltpu.sync_copy(rows.at[pl.ds(0, rem)],
                        acc.at[pl.ds(base + (RPT // K) * K, rem)])
    plsc.subcore_barrier()

    def superchunk(sc, carry):
        row0 = s * NCH + sc * SCH
        pltpu.sync_copy(srcg.at[pl.ds(row0, SCH)], sidx)
        pltpu.sync_copy(dstg.at[pl.ds(row0, SCH)], didx)
        pltpu.sync_copy(wg.at[pl.ds(row0, SCH)], wall)

        def chunk(it, cc):
            @pl.when(c == 0)
            def _():
                pltpu.async_copy(sa.at[sidx.at[it]], rows, gsem).wait()

            @pl.when(c == 1)
            def _():
                pltpu.async_copy(sb.at[sidx.at[it]], rows, gsem).wait()

            def scale(jo, c2):
                wv = wall[it, pl.ds(jo * 16, 16)]
                for j in range(16):
                    e = jo * 16 + j
                    wsc = wv[j]
                    for l in range(DH // 16):
                        rows[e, pl.ds(l * 16, 16)] = (
                            rows[e, pl.ds(l * 16, 16)] * wsc)
                return c2
            lax.fori_loop(0, K // 16, scale, 0)

            pltpu.sync_copy(rows, acc.at[didx.at[it]], add=True)
            return cc
        lax.fori_loop(0, SCH, chunk, 0)
        return carry
    lax.fori_loop(0, NCH // SCH, superchunk, 0)

    plsc.subcore_barrier()

    @pl.when(c == 0)
    def _():
        pltpu.sync_copy(acc.at[pl.ds(base, RPT)], axa.at[pl.ds(base, RPT)])

    @pl.when(c == 1)
    def _():
        pltpu.sync_copy(acc.at[pl.ds(base, RPT)], axb.at[pl.ds(base, RPT)])


_spmm = functools.partial(
    pl.kernel,
    out_type=(jax.ShapeDtypeStruct((NPAD, DH), jnp.float32),
              jax.ShapeDtypeStruct((NPAD, DH), jnp.float32)),
    mesh=plsc.VectorSubcoreMesh(core_axis_name="c", subcore_axis_name="s"),
    scratch_types=[
        pltpu.VMEM_SHARED((NPAD, DH), jnp.float32),
        pltpu.VMEM((SCH, K), jnp.int32),
        pltpu.VMEM((SCH, K), jnp.int32),
        pltpu.VMEM((SCH, K), jnp.float32),
        pltpu.VMEM((K, DH), jnp.float32),
        pltpu.SemaphoreType.DMA,
    ],
)(_spmm_body)


BN = 1024
GRID = NPAD // BN


def _update_body(sa, sb, axa, axb, xr, al, hr, wih, whh, bih, bhh, wmat,
                 osa, osb, oal):
    st = jnp.concatenate([sa[...], sb[...]], axis=1)
    z = jnp.dot(st, wih[...].T, preferred_element_type=jnp.float32)
    z = z + jnp.dot(hr[...], whh[...].T, preferred_element_type=jnp.float32)
    z = z + bih[...] + bhh[...]
    r = jnp.tanh(z)
    alpha_new = al[...] * r[:, 0:1] + r[:, 1:2]
    alph = jax.nn.sigmoid(alpha_new)
    xw = jnp.dot(st, wmat[...], preferred_element_type=jnp.float32)
    ax = jnp.concatenate([axa[...], axb[...]], axis=1)
    xv = xr[...]
    x0 = jnp.concatenate([xv, jnp.zeros_like(xv)], axis=1)
    f = alph * 0.5 * (ax - st) + xw - st + x0
    st2 = st + DT * f
    osa[...] = st2[:, :DH]
    osb[...] = st2[:, DH:]
    oal[...] = alpha_new


def _update(sa, sb, axa, axb, x, alpha, h, wih, whh, bih, bhh, wmat):
    row_spec = pl.BlockSpec((BN, DH), lambda i: (i, 0))
    return pl.pallas_call(
        _update_body,
        grid=(GRID,),
        in_specs=[
            row_spec, row_spec, row_spec, row_spec, row_spec,
            pl.BlockSpec((BN, 1), lambda i: (i, 0)),
            pl.BlockSpec((BN, 2), lambda i: (i, 0)),
            pl.BlockSpec((2, 2 * DH), lambda i: (0, 0)),
            pl.BlockSpec((2, 2), lambda i: (0, 0)),
            pl.BlockSpec((1, 2), lambda i: (0, 0)),
            pl.BlockSpec((1, 2), lambda i: (0, 0)),
            pl.BlockSpec((2 * DH, 2 * DH), lambda i: (0, 0)),
        ],
        out_specs=[
            row_spec, row_spec,
            pl.BlockSpec((BN, 1), lambda i: (i, 0)),
        ],
        out_shape=[
            jax.ShapeDtypeStruct((NPAD, DH), jnp.float32),
            jax.ShapeDtypeStruct((NPAD, DH), jnp.float32),
            jax.ShapeDtypeStruct((NPAD, 1), jnp.float32),
        ],
        compiler_params=pltpu.CompilerParams(
            dimension_semantics=("arbitrary",)),
    )(sa, sb, axa, axb, x, alpha, h, wih, whh, bih, bhh, wmat)


def _wmat_body(wr, dr, o):
    dcl = jnp.clip(dr[...], 0.0, 1.0)
    wv = wr[...]
    o[...] = jnp.dot(wv * dcl, wv.T, preferred_element_type=jnp.float32)


def _wmat(w, dvec):
    return pl.pallas_call(
        _wmat_body,
        out_shape=jax.ShapeDtypeStruct((2 * DH, 2 * DH), jnp.float32),
    )(w, dvec.reshape(1, 2 * DH))


def kernel(x, edge_index, edge_weight, rnn_w_ih, rnn_w_hh, rnn_b_ih,
           rnn_b_hh, h, alpha0, w, dvec):
    npad = NPAD - N
    x = jnp.pad(x.astype(jnp.float32), ((0, npad), (0, 0)))
    src = edge_index[1].astype(jnp.int32)
    dst = edge_index[0].astype(jnp.int32)
    ew = edge_weight.astype(jnp.float32)

    pad = EPAD - E
    srcg = jnp.pad(src, (0, pad)).reshape(NT * NCH, K)
    dstg = jnp.pad(dst, (0, pad)).reshape(NT * NCH, K)
    wg = jnp.pad(ew, (0, pad)).reshape(NT * NCH, K)

    wih = rnn_w_ih.astype(jnp.float32)
    whh = rnn_w_hh.astype(jnp.float32)
    bih = rnn_b_ih.astype(jnp.float32).reshape(1, 2)
    bhh = rnn_b_hh.astype(jnp.float32).reshape(1, 2)
    hf = jnp.pad(h.astype(jnp.float32), ((0, npad), (0, 0)))
    wmat = _wmat(w.astype(jnp.float32), dvec.astype(jnp.float32))

    sa0 = x
    sb0 = jnp.zeros_like(x)
    al0 = jnp.pad(alpha0.astype(jnp.float32), (0, npad)).reshape(NPAD, 1)

    def step(_, carry):
        sa, sb, al = carry
        axa, axb = _spmm(sa, sb, srcg, dstg, wg)
        sa, sb, al = _update(sa, sb, axa, axb, x, al, hf,
                             wih, whh, bih, bhh, wmat)
        return (sa, sb, al)

    sa, sb, al = lax.fori_loop(0, NSTEPS, step, (sa0, sb0, al0))
    return sa[:N]


# speedup vs baseline: 3.0367x; 1.2402x over previous
import functools

import jax
import jax.numpy as jnp
from jax import lax
from jax.experimental import pallas as pl
from jax.experimental.pallas import tpu as pltpu
from jax.experimental.pallas import tpu_sc as plsc

N = 10000
DH = 128
E = 320000
NSTEPS = 20
DT = 0.9 / NSTEPS

NT = 16
K = 128
NCH = 160
EPT = NCH * K
EPAD = NT * EPT
NPAD = 10240
RPT = NPAD // NT
SCH = 16


def _spmm_body(sa, sb, srcg, dstg, wg, axa, axb,
               acc, sidx, didx, wall, rows0, rows1,
               gsem0, gsem1, ssem0, ssem1):
    c = lax.axis_index("c")
    s = lax.axis_index("s")
    rows = (rows0, rows1)
    gsem = (gsem0, gsem1)
    ssem = (ssem0, ssem1)

    def zrow(e, carry):
        for l in range(DH // 16):
            rows0[e, pl.ds(l * 16, 16)] = jnp.zeros((16,), jnp.float32)
        return carry
    lax.fori_loop(0, K, zrow, 0)

    base = s * RPT
    for j in range(RPT // K):
        pltpu.sync_copy(rows0, acc.at[pl.ds(base + j * K, K)])
    plsc.subcore_barrier()

    def superchunk(sc, carry):
        row0 = s * NCH + sc * SCH
        pltpu.sync_copy(srcg.at[pl.ds(row0, SCH)], sidx)
        pltpu.sync_copy(dstg.at[pl.ds(row0, SCH)], didx)
        pltpu.sync_copy(wg.at[pl.ds(row0, SCH)], wall)

        def start_gather(it, b):
            @pl.when(c == 0)
            def _():
                pltpu.async_copy(sa.at[sidx.at[it]], rows[b], gsem[b])

            @pl.when(c == 1)
            def _():
                pltpu.async_copy(sb.at[sidx.at[it]], rows[b], gsem[b])

        def wait_gather(b):
            pltpu.make_async_copy(sa.at[sidx.at[0]], rows[b], gsem[b]).wait()

        def scale(it, b):
            rb = rows[b]

            def body(jo, c2):
                wv = wall[it, pl.ds(jo * 16, 16)]
                for j in range(16):
                    e = jo * 16 + j
                    wsc = wv[j]
                    for l in range(DH // 16):
                        rb[e, pl.ds(l * 16, 16)] = (
                            rb[e, pl.ds(l * 16, 16)] * wsc)
                return c2
            lax.fori_loop(0, K // 16, body, 0)

        scatters = {}
        start_gather(0, 0)
        for it in range(SCH):
            b = it % 2
            if it + 1 < SCH:
                b2 = (it + 1) % 2
                if it - 1 >= 0:
                    scatters[it - 1].wait()
                start_gather(it + 1, b2)
            wait_gather(b)
            scale(it, b)
            scatters[it] = pltpu.async_copy(
                rows[b], acc.at[didx.at[it]], ssem[b], add=True)
        scatters[SCH - 2].wait()
        scatters[SCH - 1].wait()
        return carry
    lax.fori_loop(0, NCH // SCH, superchunk, 0)

    plsc.subcore_barrier()

    @pl.when(c == 0)
    def _():
        pltpu.sync_copy(acc.at[pl.ds(base, RPT)], axa.at[pl.ds(base, RPT)])

    @pl.when(c == 1)
    def _():
        pltpu.sync_copy(acc.at[pl.ds(base, RPT)], axb.at[pl.ds(base, RPT)])


_spmm = functools.partial(
    pl.kernel,
    out_type=(jax.ShapeDtypeStruct((NPAD, DH), jnp.float32),
              jax.ShapeDtypeStruct((NPAD, DH), jnp.float32)),
    mesh=plsc.VectorSubcoreMesh(core_axis_name="c", subcore_axis_name="s"),
    scratch_types=[
        pltpu.VMEM_SHARED((NPAD, DH), jnp.float32),
        pltpu.VMEM((SCH, K), jnp.int32),
        pltpu.VMEM((SCH, K), jnp.int32),
        pltpu.VMEM((SCH, K), jnp.float32),
        pltpu.VMEM((K, DH), jnp.float32),
        pltpu.VMEM((K, DH), jnp.float32),
        pltpu.SemaphoreType.DMA,
        pltpu.SemaphoreType.DMA,
        pltpu.SemaphoreType.DMA,
        pltpu.SemaphoreType.DMA,
    ],
)(_spmm_body)


BN = 1024
GRID = NPAD // BN


def _update_body(sa, sb, axa, axb, xr, al, hr, wih, whh, bih, bhh, wmat,
                 osa, osb, oal):
    st = jnp.concatenate([sa[...], sb[...]], axis=1)
    z = jnp.dot(st, wih[...].T, preferred_element_type=jnp.float32)
    z = z + jnp.dot(hr[...], whh[...].T, preferred_element_type=jnp.float32)
    z = z + bih[...] + bhh[...]
    r = jnp.tanh(z)
    alpha_new = al[...] * r[:, 0:1] + r[:, 1:2]
    alph = jax.nn.sigmoid(alpha_new)
    xw = jnp.dot(st, wmat[...], preferred_element_type=jnp.float32)
    ax = jnp.concatenate([axa[...], axb[...]], axis=1)
    xv = xr[...]
    x0 = jnp.concatenate([xv, jnp.zeros_like(xv)], axis=1)
    f = alph * 0.5 * (ax - st) + xw - st + x0
    st2 = st + DT * f
    osa[...] = st2[:, :DH]
    osb[...] = st2[:, DH:]
    oal[...] = alpha_new


def _update(sa, sb, axa, axb, x, alpha, h, wih, whh, bih, bhh, wmat):
    row_spec = pl.BlockSpec((BN, DH), lambda i: (i, 0))
    return pl.pallas_call(
        _update_body,
        grid=(GRID,),
        in_specs=[
            row_spec, row_spec, row_spec, row_spec, row_spec,
            pl.BlockSpec((BN, 1), lambda i: (i, 0)),
            pl.BlockSpec((BN, 2), lambda i: (i, 0)),
            pl.BlockSpec((2, 2 * DH), lambda i: (0, 0)),
            pl.BlockSpec((2, 2), lambda i: (0, 0)),
            pl.BlockSpec((1, 2), lambda i: (0, 0)),
            pl.BlockSpec((1, 2), lambda i: (0, 0)),
            pl.BlockSpec((2 * DH, 2 * DH), lambda i: (0, 0)),
        ],
        out_specs=[
            row_spec, row_spec,
            pl.BlockSpec((BN, 1), lambda i: (i, 0)),
        ],
        out_shape=[
            jax.ShapeDtypeStruct((NPAD, DH), jnp.float32),
            jax.ShapeDtypeStruct((NPAD, DH), jnp.float32),
            jax.ShapeDtypeStruct((NPAD, 1), jnp.float32),
        ],
        compiler_params=pltpu.CompilerParams(
            dimension_semantics=("arbitrary",)),
    )(sa, sb, axa, axb, x, alpha, h, wih, whh, bih, bhh, wmat)


def _wmat_body(wr, dr, o):
    dcl = jnp.clip(dr[...], 0.0, 1.0)
    wv = wr[...]
    o[...] = jnp.dot(wv * dcl, wv.T, preferred_element_type=jnp.float32)


def _wmat(w, dvec):
    return pl.pallas_call(
        _wmat_body,
        out_shape=jax.ShapeDtypeStruct((2 * DH, 2 * DH), jnp.float32),
    )(w, dvec.reshape(1, 2 * DH))


def kernel(x, edge_index, edge_weight, rnn_w_ih, rnn_w_hh, rnn_b_ih,
           rnn_b_hh, h, alpha0, w, dvec):
    npad = NPAD - N
    x = jnp.pad(x.astype(jnp.float32), ((0, npad), (0, 0)))
    src = edge_index[1].astype(jnp.int32)
    dst = edge_index[0].astype(jnp.int32)
    ew = edge_weight.astype(jnp.float32)

    pad = EPAD - E
    srcg = jnp.pad(src, (0, pad)).reshape(NT * NCH, K)
    dstg = jnp.pad(dst, (0, pad)).reshape(NT * NCH, K)
    wg = jnp.pad(ew, (0, pad)).reshape(NT * NCH, K)

    wih = rnn_w_ih.astype(jnp.float32)
    whh = rnn_w_hh.astype(jnp.float32)
    bih = rnn_b_ih.astype(jnp.float32).reshape(1, 2)
    bhh = rnn_b_hh.astype(jnp.float32).reshape(1, 2)
    hf = jnp.pad(h.astype(jnp.float32), ((0, npad), (0, 0)))
    wmat = _wmat(w.astype(jnp.float32), dvec.astype(jnp.float32))

    sa0 = x
    sb0 = jnp.zeros_like(x)
    al0 = jnp.pad(alpha0.astype(jnp.float32), (0, npad)).reshape(NPAD, 1)

    def step(_, carry):
        sa, sb, al = carry
        axa, axb = _spmm(sa, sb, srcg, dstg, wg)
        sa, sb, al = _update(sa, sb, axa, axb, x, al, hf,
                             wih, whh, bih, bhh, wmat)
        return (sa, sb, al)

    sa, sb, al = lax.fori_loop(0, NSTEPS, step, (sa0, sb0, al0))
    return sa[:N]
